# baseline (speedup 1.0000x reference)
import functools

import jax
import jax.numpy as jnp
from jax import lax
from jax.experimental import pallas as pl
from jax.experimental.pallas import tpu as pltpu
from jax.experimental.pallas import tpu_sc as plsc

VOCAB = 128000
EMB = 2048
HID = 10240
EPS = 1e-5
BATCH = 4
SEQ = 2048
NTOK = BATCH * SEQ

_NSLICE = 4
_SLICE = NTOK // _NSLICE


_SC_NC = 2
_SC_NS = 16
_SC_NW = _SC_NC * _SC_NS
_CH = 16


def _gather_sc(ids2d, table, nrows):
    rows_per_w = nrows // _SC_NW
    nch = rows_per_w // _CH
    mesh = plsc.VectorSubcoreMesh(core_axis_name="c", subcore_axis_name="s")

    @functools.partial(
        pl.kernel,
        mesh=mesh,
        out_type=jax.ShapeDtypeStruct((nrows, EMB), jnp.float32),
        scratch_types=[
            pltpu.VMEM((nch, _CH), jnp.int32),
            pltpu.VMEM((2, _CH, EMB), jnp.float32),
            pltpu.SemaphoreType.DMA,
            pltpu.SemaphoreType.DMA,
            pltpu.SemaphoreType.DMA,
            pltpu.SemaphoreType.DMA,
        ],
    )
    def k(ids_hbm, table_hbm, out_hbm, idx_v, rows_v, gs0, gs1, ws0, ws1):
        wid = lax.axis_index("s") * _SC_NC + lax.axis_index("c")
        base = wid * rows_per_w
        pltpu.sync_copy(ids_hbm.at[pl.ds(wid * nch, nch)], idx_v)
        gsem = [gs0, gs1]
        wsem = [ws0, ws1]
        gcp = [None, None]
        wcp = [None, None]

        def start_gather(j):
            b = j % 2
            gcp[b] = pltpu.async_copy(
                table_hbm.at[idx_v.at[j]], rows_v.at[b], gsem[b])

        start_gather(0)
        for j in range(nch):
            b = j % 2
            gcp[b].wait()
            if j + 1 < nch:
                if j >= 1:
                    wcp[1 - b].wait()
                start_gather(j + 1)
            wcp[b] = pltpu.async_copy(
                rows_v.at[b], out_hbm.at[pl.ds(base + j * _CH, _CH)], wsem[b])
        wcp[nch % 2].wait()
        wcp[(nch - 1) % 2].wait()

    return k(ids2d, table)


LN_BM = 512


def _ln_body(x_ref, g_ref, b_ref, o_ref):
    x = x_ref[...]
    mean = jnp.mean(x, axis=-1, keepdims=True)
    xc = x - mean
    var = jnp.mean(xc * xc, axis=-1, keepdims=True)
    xhat = xc * jax.lax.rsqrt(var + EPS)
    y = xhat * g_ref[...] + b_ref[...]
    o_ref[...] = y.astype(jnp.bfloat16)


def _layernorm_tc(emb, gamma2d, beta2d, nrows):
    return pl.pallas_call(
        _ln_body,
        grid=(nrows // LN_BM,),
        in_specs=[
            pl.BlockSpec((LN_BM, EMB), lambda i: (i, 0)),
            pl.BlockSpec((1, EMB), lambda i: (0, 0)),
            pl.BlockSpec((1, EMB), lambda i: (0, 0)),
        ],
        out_specs=pl.BlockSpec((LN_BM, EMB), lambda i: (i, 0)),
        out_shape=jax.ShapeDtypeStruct((nrows, EMB), jnp.bfloat16),
    )(emb, gamma2d, beta2d)


MM_BM = 2048
MM_BN = 1024


def _mm_body(h_ref, w_ref, b_ref, o_ref):
    acc = jnp.dot(h_ref[...], w_ref[...], preferred_element_type=jnp.float32)
    o_ref[...] = acc + b_ref[...]


def _matmul_tc(h, w_bf16, bias2d, nrows):
    return pl.pallas_call(
        _mm_body,
        grid=(nrows // MM_BM, HID // MM_BN),
        in_specs=[
            pl.BlockSpec((MM_BM, EMB), lambda m, n: (m, 0)),
            pl.BlockSpec((EMB, MM_BN), lambda m, n: (0, n)),
            pl.BlockSpec((1, MM_BN), lambda m, n: (0, n)),
        ],
        out_specs=pl.BlockSpec((MM_BM, MM_BN), lambda m, n: (m, n)),
        out_shape=jax.ShapeDtypeStruct((nrows, HID), jnp.float32),
    )(h, w_bf16, bias2d)


@jax.jit
def kernel(input_ids, table, ln_gamma, ln_beta, proj_w, proj_b):
    w_bf16 = proj_w.astype(jnp.bfloat16)
    bias2d = proj_b.reshape(1, HID)
    gamma2d = ln_gamma.reshape(1, EMB)
    beta2d = ln_beta.reshape(1, EMB)
    nch = _SLICE // _SC_NW // _CH
    ids3d = input_ids.reshape(_NSLICE, _SC_NW * nch, _CH).astype(jnp.int32)
    outs = []
    for s in range(_NSLICE):
        emb = _gather_sc(ids3d[s], table, _SLICE)
        h = _layernorm_tc(emb, gamma2d, beta2d, _SLICE)
        outs.append(_matmul_tc(h, w_bf16, bias2d, _SLICE))
    out = jnp.concatenate(outs, axis=0)
    return out.reshape(BATCH, SEQ, HID)


# speedup vs baseline: 1.4348x; 1.4348x over previous
import functools

import jax
import jax.numpy as jnp
from jax import lax
from jax.experimental import pallas as pl
from jax.experimental.pallas import tpu as pltpu
from jax.experimental.pallas import tpu_sc as plsc

VOCAB = 128000
EMB = 2048
HID = 10240
EPS = 1e-5
BATCH = 4
SEQ = 2048
NTOK = BATCH * SEQ

_NSLICE = 4
_SLICE = NTOK // _NSLICE


_SC_NC = 2
_SC_NS = 16
_SC_NW = _SC_NC * _SC_NS
_CH = 16


def _gather_sc(ids2d, table, nrows):
    rows_per_w = nrows // _SC_NW
    nch = rows_per_w // _CH
    mesh = plsc.VectorSubcoreMesh(core_axis_name="c", subcore_axis_name="s")

    @functools.partial(
        pl.kernel,
        mesh=mesh,
        out_type=jax.ShapeDtypeStruct((nrows, EMB), jnp.float32),
        scratch_types=[
            pltpu.VMEM((nch, _CH), jnp.int32),
            pltpu.VMEM((2, _CH, EMB), jnp.float32),
            pltpu.SemaphoreType.DMA,
            pltpu.SemaphoreType.DMA,
            pltpu.SemaphoreType.DMA,
            pltpu.SemaphoreType.DMA,
        ],
    )
    def k(ids_hbm, table_hbm, out_hbm, idx_v, rows_v, gs0, gs1, ws0, ws1):
        wid = lax.axis_index("s") * _SC_NC + lax.axis_index("c")
        base = wid * rows_per_w
        pltpu.sync_copy(ids_hbm.at[pl.ds(wid * nch, nch)], idx_v)
        gsem = [gs0, gs1]
        wsem = [ws0, ws1]
        gcp = [None, None]
        wcp = [None, None]

        def start_gather(j):
            b = j % 2
            gcp[b] = pltpu.async_copy(
                table_hbm.at[idx_v.at[j]], rows_v.at[b], gsem[b])

        start_gather(0)
        for j in range(nch):
            b = j % 2
            gcp[b].wait()
            if j + 1 < nch:
                if j >= 1:
                    wcp[1 - b].wait()
                start_gather(j + 1)
            wcp[b] = pltpu.async_copy(
                rows_v.at[b], out_hbm.at[pl.ds(base + j * _CH, _CH)], wsem[b])
        wcp[nch % 2].wait()
        wcp[(nch - 1) % 2].wait()

    return k(ids2d, table)


MM_BN = 512


def _mm_ln_body(emb_ref, w_ref, b_ref, g_ref, bt_ref, *rest):
    o_ref, h_ref = rest[-2], rest[-1]

    @pl.when(pl.program_id(0) == 0)
    def _ln():
        x = emb_ref[...]
        mean = jnp.mean(x, axis=-1, keepdims=True)
        xc = x - mean
        var = jnp.mean(xc * xc, axis=-1, keepdims=True)
        xhat = xc * jax.lax.rsqrt(var + EPS)
        h_ref[...] = (xhat * g_ref[...] + bt_ref[...]).astype(jnp.bfloat16)

    acc = jnp.dot(h_ref[...], w_ref[...], preferred_element_type=jnp.float32)
    o_ref[...] = acc + b_ref[...]


def _mm_ln_slice(emb_s, w_bf16, bias2d, gamma2d, beta2d, prev_out, s):
    in_specs = [
        pl.BlockSpec((_SLICE, EMB), lambda n: (0, 0)),
        pl.BlockSpec((EMB, MM_BN), lambda n: (0, n)),
        pl.BlockSpec((1, MM_BN), lambda n: (0, n)),
        pl.BlockSpec((1, EMB), lambda n: (0, 0)),
        pl.BlockSpec((1, EMB), lambda n: (0, 0)),
    ]
    args = [emb_s, w_bf16, bias2d, gamma2d, beta2d]
    aliases = {}
    if prev_out is not None:
        in_specs.append(pl.BlockSpec(memory_space=pl.ANY))
        args.append(prev_out)
        aliases = {5: 0}
    return pl.pallas_call(
        _mm_ln_body,
        grid=(HID // MM_BN,),
        in_specs=in_specs,
        out_specs=pl.BlockSpec((_SLICE, MM_BN), lambda n: (s, n)),
        out_shape=jax.ShapeDtypeStruct((NTOK, HID), jnp.float32),
        scratch_shapes=[pltpu.VMEM((_SLICE, EMB), jnp.bfloat16)],
        input_output_aliases=aliases,
        compiler_params=pltpu.CompilerParams(
            dimension_semantics=("arbitrary",)),
    )(*args)


@jax.jit
def kernel(input_ids, table, ln_gamma, ln_beta, proj_w, proj_b):
    w_bf16 = proj_w.astype(jnp.bfloat16)
    bias2d = proj_b.reshape(1, HID)
    gamma2d = ln_gamma.reshape(1, EMB)
    beta2d = ln_beta.reshape(1, EMB)
    nch = _SLICE // _SC_NW // _CH
    ids3d = input_ids.reshape(_NSLICE, _SC_NW * nch, _CH).astype(jnp.int32)
    embs = [_gather_sc(ids3d[s], table, _SLICE) for s in range(_NSLICE)]
    out = None
    for s in range(_NSLICE):
        out = _mm_ln_slice(embs[s], w_bf16, bias2d, gamma2d, beta2d, out, s)
    return out.reshape(BATCH, SEQ, HID)


# speedup vs baseline: 1.4724x; 1.0262x over previous
import functools

import jax
import jax.numpy as jnp
from jax import lax
from jax.experimental import pallas as pl
from jax.experimental.pallas import tpu as pltpu
from jax.experimental.pallas import tpu_sc as plsc

VOCAB = 128000
EMB = 2048
HID = 10240
EPS = 1e-5
BATCH = 4
SEQ = 2048
NTOK = BATCH * SEQ


_SC_NC = 2
_SC_NS = 16
_SC_NW = _SC_NC * _SC_NS
_CH = 16
_NBUF = 3


def _gather_sc(ids2d, table, nrows):
    rows_per_w = nrows // _SC_NW
    nch = rows_per_w // _CH
    mesh = plsc.VectorSubcoreMesh(core_axis_name="c", subcore_axis_name="s")

    @functools.partial(
        pl.kernel,
        mesh=mesh,
        out_type=jax.ShapeDtypeStruct((nrows, EMB), jnp.float32),
        scratch_types=(
            [pltpu.VMEM((nch, _CH), jnp.int32),
             pltpu.VMEM((_NBUF, _CH, EMB), jnp.float32)]
            + [pltpu.SemaphoreType.DMA] * (2 * _NBUF)
        ),
    )
    def k(ids_hbm, table_hbm, out_hbm, idx_v, rows_v, *sems):
        gsem = sems[:_NBUF]
        wsem = sems[_NBUF:]
        wid = lax.axis_index("s") * _SC_NC + lax.axis_index("c")
        base = wid * rows_per_w
        pltpu.sync_copy(ids_hbm.at[pl.ds(wid * nch, nch)], idx_v)
        gcp = [None] * _NBUF
        wcp = [None] * _NBUF

        def start_gather(j):
            b = j % _NBUF
            gcp[b] = pltpu.async_copy(
                table_hbm.at[idx_v.at[j]], rows_v.at[b], gsem[b])

        start_gather(0)
        if nch > 1:
            start_gather(1)
        for j in range(nch):
            b = j % _NBUF
            gcp[b].wait()
            if j + 2 < nch:
                if j >= 1:
                    wcp[(j - 1) % _NBUF].wait()
                start_gather(j + 2)
            wcp[b] = pltpu.async_copy(
                rows_v.at[b], out_hbm.at[pl.ds(base + j * _CH, _CH)], wsem[b])
        for j in range(max(0, nch - _NBUF), nch):
            wcp[j % _NBUF].wait()

    return k(ids2d, table)


LN_BM = 512


def _ln_body(x_ref, g_ref, b_ref, o_ref):
    x = x_ref[...]
    mean = jnp.mean(x, axis=-1, keepdims=True)
    xc = x - mean
    var = jnp.mean(xc * xc, axis=-1, keepdims=True)
    xhat = xc * jax.lax.rsqrt(var + EPS)
    y = xhat * g_ref[...] + b_ref[...]
    o_ref[...] = y.astype(jnp.bfloat16)


def _layernorm_tc(emb, gamma2d, beta2d, nrows):
    return pl.pallas_call(
        _ln_body,
        grid=(nrows // LN_BM,),
        in_specs=[
            pl.BlockSpec((LN_BM, EMB), lambda i: (i, 0)),
            pl.BlockSpec((1, EMB), lambda i: (0, 0)),
            pl.BlockSpec((1, EMB), lambda i: (0, 0)),
        ],
        out_specs=pl.BlockSpec((LN_BM, EMB), lambda i: (i, 0)),
        out_shape=jax.ShapeDtypeStruct((nrows, EMB), jnp.bfloat16),
    )(emb, gamma2d, beta2d)


MM_BM = 2048
MM_BN = 1024


def _mm_body(h_ref, w_ref, b_ref, o_ref):
    acc = jnp.dot(h_ref[...], w_ref[...], preferred_element_type=jnp.float32)
    o_ref[...] = acc + b_ref[...]


def _matmul_tc(h, w_bf16, bias2d, nrows):
    return pl.pallas_call(
        _mm_body,
        grid=(nrows // MM_BM, HID // MM_BN),
        in_specs=[
            pl.BlockSpec((MM_BM, EMB), lambda m, n: (m, 0)),
            pl.BlockSpec((EMB, MM_BN), lambda m, n: (0, n)),
            pl.BlockSpec((1, MM_BN), lambda m, n: (0, n)),
        ],
        out_specs=pl.BlockSpec((MM_BM, MM_BN), lambda m, n: (m, n)),
        out_shape=jax.ShapeDtypeStruct((nrows, HID), jnp.float32),
    )(h, w_bf16, bias2d)


@jax.jit
def kernel(input_ids, table, ln_gamma, ln_beta, proj_w, proj_b):
    w_bf16 = proj_w.astype(jnp.bfloat16)
    bias2d = proj_b.reshape(1, HID)
    gamma2d = ln_gamma.reshape(1, EMB)
    beta2d = ln_beta.reshape(1, EMB)
    nch = NTOK // _SC_NW // _CH
    ids2d = input_ids.reshape(_SC_NW * nch, _CH).astype(jnp.int32)
    emb = _gather_sc(ids2d, table, NTOK)
    h = _layernorm_tc(emb, gamma2d, beta2d, NTOK)
    out = _matmul_tc(h, w_bf16, bias2d, NTOK)
    return out.reshape(BATCH, SEQ, HID)
